# baseline (speedup 1.0000x reference)
import functools

import jax
import jax.numpy as jnp
from jax import lax
from jax.experimental import pallas as pl
from jax.experimental.pallas import tpu as pltpu
from jax.experimental.pallas import tpu_sc as plsc

NB, NT, NP, D = 4, 32, 196, 768
L = 16
DV = D // L
ROWS = 49
NCHUNK = NP // ROWS
SLAB = NP * D
CHUNK = ROWS * D
Q = NB * NCHUNK
NBUF = 3


def _body(x_hbm, fe_hbm, out_hbm, pe_v, b0, b1, b2, si0, si1, si2, so0, so1, so2):
    bufs = [b0, b1, b2]
    sin = [si0, si1, si2]
    sout = [so0, so1, so2]

    c = lax.axis_index("c")
    s = lax.axis_index("s")
    w = s * 2 + c

    pltpu.sync_copy(fe_hbm.at[pl.ds(w * D, D)], pe_v)

    def chunk_base(q):
        b, ch = divmod(q, NCHUNK)
        return (b * NT + 0) * SLAB + w * SLAB + ch * CHUNK

    out_handles = [None] * NBUF

    def start_in(q):
        bb = q % NBUF
        return pltpu.make_async_copy(
            x_hbm.at[pl.ds(chunk_base(q), CHUNK)], bufs[bb], sin[bb])

    h_in = start_in(0)
    h_in.start()
    pending_in = h_in

    for q in range(Q):
        bb = q % NBUF
        nq = q + 1
        next_in = None
        if nq < Q:
            nbb = nq % NBUF
            if out_handles[nbb] is not None:
                out_handles[nbb].wait()
                out_handles[nbb] = None
            next_in = start_in(nq)
            next_in.start()
        pending_in.wait()
        pending_in = next_in

        buf = bufs[bb]

        def row_body(r, carry):
            r0 = r * D
            for j in range(DV):
                plsc.addupdate(buf.at[pl.ds(r0 + j * L, L)],
                               pe_v[pl.ds(j * L, L)])
            return carry

        lax.fori_loop(0, ROWS, row_body, 0)

        h_out = pltpu.make_async_copy(
            buf, out_hbm.at[pl.ds(chunk_base(q), CHUNK)], sout[bb])
        h_out.start()
        out_handles[bb] = h_out

    for h in out_handles:
        if h is not None:
            h.wait()


@jax.jit
def _run(x, frame_embed):
    mesh = plsc.VectorSubcoreMesh(core_axis_name="c", subcore_axis_name="s")
    k = functools.partial(
        pl.kernel,
        mesh=mesh,
        out_type=jax.ShapeDtypeStruct((NB * NT * NP * D,), jnp.float32),
        scratch_types=(
            [pltpu.VMEM((D,), jnp.float32)]
            + [pltpu.VMEM((CHUNK,), jnp.float32) for _ in range(NBUF)]
            + [pltpu.SemaphoreType.DMA for _ in range(2 * NBUF)]
        ),
    )(_body)
    out = k(x.reshape(-1), frame_embed.reshape(-1))
    return out.reshape(NB, NT, NP, D)


def kernel(x, frame_embed):
    return _run(x, frame_embed)


# speedup vs baseline: 2.1879x; 2.1879x over previous
import functools

import jax
import jax.numpy as jnp
from jax import lax
from jax.experimental import pallas as pl
from jax.experimental.pallas import tpu as pltpu
from jax.experimental.pallas import tpu_sc as plsc

NB, NT, NP, D = 4, 32, 196, 768
L = 16
CW = 128
CV = CW // L
NCOL = D // CW
Q = NB * NCOL
NBUF = 3


def _body(x_hbm, fe_hbm, out_hbm, fe_v, b0, b1, b2, si0, si1, si2, so0, so1, so2):
    bufs = [b0, b1, b2]
    sin = [si0, si1, si2]
    sout = [so0, so1, so2]

    c = lax.axis_index("c")
    s = lax.axis_index("s")
    w = s * 2 + c

    pltpu.sync_copy(fe_hbm, fe_v)

    def chunk_slice(ref, q):
        b, j = divmod(q, NCOL)
        return ref.at[b, w, :, pl.ds(j * CW, CW)]

    out_handles = [None] * NBUF

    def start_in(q):
        bb = q % NBUF
        h = pltpu.make_async_copy(chunk_slice(x_hbm, q), bufs[bb], sin[bb])
        h.start()
        return h

    pending_in = start_in(0)

    for q in range(Q):
        bb = q % NBUF
        j = q % NCOL
        next_in = None
        if q + 1 < Q:
            nbb = (q + 1) % NBUF
            if out_handles[nbb] is not None:
                out_handles[nbb].wait()
                out_handles[nbb] = None
            next_in = start_in(q + 1)
        pending_in.wait()
        pending_in = next_in

        buf = bufs[bb]
        pe = [fe_v[w, pl.ds(j * CW + k * L, L)] for k in range(CV)]

        @plsc.parallel_loop(0, NP, unroll=4)
        def _row(r):
            for k in range(CV):
                plsc.addupdate(buf.at[r, pl.ds(k * L, L)], pe[k])

        h_out = pltpu.make_async_copy(buf, chunk_slice(out_hbm, q), sout[bb])
        h_out.start()
        out_handles[bb] = h_out

    for h in out_handles:
        if h is not None:
            h.wait()


@jax.jit
def _run(x, frame_embed):
    mesh = plsc.VectorSubcoreMesh(core_axis_name="c", subcore_axis_name="s")
    k = functools.partial(
        pl.kernel,
        mesh=mesh,
        out_type=jax.ShapeDtypeStruct((NB, NT, NP, D), jnp.float32),
        scratch_types=(
            [pltpu.VMEM((NT, D), jnp.float32)]
            + [pltpu.VMEM((NP, CW), jnp.float32) for _ in range(NBUF)]
            + [pltpu.SemaphoreType.DMA for _ in range(2 * NBUF)]
        ),
    )(_body)
    return k(x, frame_embed)


def kernel(x, frame_embed):
    return _run(x, frame_embed)
